# baseline (speedup 1.0000x reference)
import functools

import jax
import jax.numpy as jnp
import numpy as np
from jax import lax
from jax.experimental import pallas as pl
from jax.experimental.pallas import tpu as pltpu
from jax.experimental.pallas import tpu_sc as plsc

D_MODEL = 32
BLK = 128
NBUF = 2
SCALE = np.float32(np.sqrt(np.float32(D_MODEL)))

_NC = 2
_NS = 16
_NW = _NC * _NS
_SUB = 8
_DB = D_MODEL // _SUB


def _make_sc_kernel(n_seq: int, seq_len: int):
    assert n_seq == _NW * BLK
    n_tiles = seq_len * _DB * (n_seq // BLK)

    mesh = plsc.VectorSubcoreMesh(core_axis_name="c", subcore_axis_name="s")

    @functools.partial(
        pl.kernel,
        mesh=mesh,
        out_type=jax.ShapeDtypeStruct((n_tiles, _SUB, BLK), jnp.float32),
        compiler_params=pltpu.CompilerParams(
            use_tc_tiling_on_sc=False, needs_layout_passes=False
        ),
        scratch_types=[
            pltpu.VMEM((seq_len, BLK), jnp.int32),
            pltpu.VMEM((NBUF, BLK, D_MODEL), jnp.float32),
            pltpu.VMEM((NBUF, _DB, _SUB, BLK), jnp.float32),
            pltpu.SemaphoreType.DMA,
            pltpu.SemaphoreType.DMA,
            pltpu.SemaphoreType.DMA,
            pltpu.SemaphoreType.DMA,
        ],
    )
    def k(idxt_hbm, table_hbm, out_hbm, idx_v, gbuf, wbuf, gs0, gs1, ws0, ws1):
        gsems = (gs0, gs1)
        wsems = (ws0, ws1)
        wid = lax.axis_index("s") * _NC + lax.axis_index("c")

        pltpu.sync_copy(idxt_hbm.at[:, pl.ds(wid * BLK, BLK)], idx_v)

        def fire_gather(t, b):
            pltpu.async_copy(table_hbm.at[idx_v.at[t]], gbuf.at[b], gsems[b])

        def fire_writes(t, b):
            for db in range(_DB):
                pltpu.async_copy(
                    wbuf.at[b, db], out_hbm.at[(t * _DB + db) * _NW + wid], wsems[b]
                )

        def wait_writes(b):
            for _ in range(_DB):
                pltpu.make_async_copy(wbuf.at[0, 0], out_hbm.at[0], wsems[b]).wait()

        for b in range(NBUF):
            fire_gather(b, b)

        lanes = lax.iota(jnp.int32, 16)

        def body(t0, carry):
            for b in range(NBUF):
                t = t0 + b
                @pl.when(t0 >= NBUF)
                def _():
                    wait_writes(b)

                pltpu.make_async_copy(
                    table_hbm.at[idx_v.at[0]], gbuf.at[b], gsems[b]
                ).wait()

                for db in range(_DB):
                    for dr in range(_SUB):
                        d = db * _SUB + dr
                        dvec = jnp.full((16,), d, jnp.int32)
                        for j in range(0, BLK, 16):
                            rows = plsc.load_gather(
                                gbuf.at[b], [lanes + j, dvec]
                            )
                            wbuf[b, db, dr, pl.ds(j, 16)] = rows * SCALE

                fire_writes(t, b)

                @pl.when(t0 + NBUF < seq_len)
                def _():
                    fire_gather(t + NBUF, b)
            return carry

        lax.fori_loop(0, seq_len // NBUF,
                      lambda i, cr: body(i * NBUF, cr), 0, unroll=False)

        for b in range(NBUF):
            wait_writes(b)

    return k


def kernel(sequences, table):
    n_seq, seq_len = sequences.shape
    idxt = sequences.T
    out_tiles = _make_sc_kernel(n_seq, seq_len)(idxt, table)
    out = out_tiles.reshape(seq_len, _DB, _NW, _SUB, BLK)
    out = out.transpose(2, 4, 0, 1, 3)
    return out.reshape(n_seq, seq_len, D_MODEL)


# speedup vs baseline: 1.5150x; 1.5150x over previous
import functools

import jax
import jax.numpy as jnp
import numpy as np
from jax import lax
from jax.experimental import pallas as pl
from jax.experimental.pallas import tpu as pltpu
from jax.experimental.pallas import tpu_sc as plsc

D_MODEL = 32
BLK = 128
NBUF = 2
SCALE = np.float32(np.sqrt(np.float32(D_MODEL)))

_NC = 2
_NS = 16
_NW = _NC * _NS
_SUB = 8
_DB = D_MODEL // _SUB


def _make_sc_kernel(n_seq: int, seq_len: int):
    assert n_seq == _NW * BLK
    n_tiles = seq_len * _DB * (n_seq // BLK)

    mesh = plsc.VectorSubcoreMesh(core_axis_name="c", subcore_axis_name="s")

    @functools.partial(
        pl.kernel,
        mesh=mesh,
        out_type=jax.ShapeDtypeStruct((n_tiles, _SUB, BLK), jnp.float32),
        compiler_params=pltpu.CompilerParams(
            use_tc_tiling_on_sc=False, needs_layout_passes=False
        ),
        scratch_types=[
            pltpu.VMEM((seq_len, BLK), jnp.int32),
            pltpu.VMEM((NBUF, BLK, D_MODEL), jnp.float32),
            pltpu.VMEM((NBUF, D_MODEL, BLK + 1), jnp.float32),
            pltpu.SemaphoreType.DMA,
            pltpu.SemaphoreType.DMA,
            pltpu.SemaphoreType.DMA,
            pltpu.SemaphoreType.DMA,
        ],
    )
    def k(idxt_hbm, table_hbm, out_hbm, idx_v, gbuf, wbuf, gs0, gs1, ws0, ws1):
        gsems = (gs0, gs1)
        wsems = (ws0, ws1)
        wid = lax.axis_index("s") * _NC + lax.axis_index("c")

        pltpu.sync_copy(idxt_hbm.at[:, pl.ds(wid * BLK, BLK)], idx_v)

        def fire_gather(t, b):
            pltpu.async_copy(table_hbm.at[idx_v.at[t]], gbuf.at[b], gsems[b])

        def fire_writes(t, b):
            for db in range(_DB):
                pltpu.async_copy(
                    wbuf.at[b, pl.ds(db * _SUB, _SUB), pl.ds(0, BLK)],
                    out_hbm.at[(t * _DB + db) * _NW + wid],
                    wsems[b],
                )

        def wait_writes(b):
            for _ in range(_DB):
                pltpu.make_async_copy(
                    wbuf.at[0, pl.ds(0, _SUB), pl.ds(0, BLK)],
                    out_hbm.at[0],
                    wsems[b],
                ).wait()

        for b in range(NBUF):
            fire_gather(b, b)

        lanes = lax.iota(jnp.int32, 16)

        def body(t0, carry):
            for b in range(NBUF):
                t = t0 + b
                @pl.when(t0 >= NBUF)
                def _():
                    wait_writes(b)

                pltpu.make_async_copy(
                    table_hbm.at[idx_v.at[0]], gbuf.at[b], gsems[b]
                ).wait()

                for tok in range(BLK):
                    svec = jnp.full((16,), tok, jnp.int32)
                    for c in (0, 16):
                        vals = gbuf[b, tok, pl.ds(c, 16)] * SCALE
                        plsc.store_scatter(
                            wbuf.at[b], [lanes + c, svec], vals
                        )

                fire_writes(t, b)

                @pl.when(t0 + NBUF < seq_len)
                def _():
                    fire_gather(t + NBUF, b)
            return carry

        lax.fori_loop(0, seq_len // NBUF,
                      lambda i, cr: body(i * NBUF, cr), 0, unroll=False)

        for b in range(NBUF):
            wait_writes(b)

    return k


def kernel(sequences, table):
    n_seq, seq_len = sequences.shape
    idxt = sequences.T
    out_tiles = _make_sc_kernel(n_seq, seq_len)(idxt, table)
    out = out_tiles.reshape(seq_len, _DB, _NW, _SUB, BLK)
    out = out.transpose(2, 4, 0, 1, 3)
    return out.reshape(n_seq, seq_len, D_MODEL)


# speedup vs baseline: 2.0439x; 1.3491x over previous
import functools

import jax
import jax.numpy as jnp
import numpy as np
from jax import lax
from jax.experimental import pallas as pl
from jax.experimental.pallas import tpu as pltpu
from jax.experimental.pallas import tpu_sc as plsc

D_MODEL = 32
BLK = 128
NBUF = 2
SCALE = np.float32(np.sqrt(np.float32(D_MODEL)))

_NC = 2
_NS = 16
_NW = _NC * _NS
_SUB = 8
_DB = D_MODEL // _SUB


def _make_sc_kernel(n_seq: int, seq_len: int):
    assert n_seq == _NW * BLK
    n_tiles = seq_len * _DB * (n_seq // BLK)

    mesh = plsc.VectorSubcoreMesh(core_axis_name="c", subcore_axis_name="s")

    @functools.partial(
        pl.kernel,
        mesh=mesh,
        out_type=jax.ShapeDtypeStruct((n_tiles, _SUB, BLK), jnp.float32),
        compiler_params=pltpu.CompilerParams(
            use_tc_tiling_on_sc=False, needs_layout_passes=False
        ),
        scratch_types=[
            pltpu.VMEM((seq_len, BLK), jnp.int32),
            pltpu.VMEM((NBUF, BLK, D_MODEL), jnp.float32),
            pltpu.VMEM((NBUF, D_MODEL, BLK + 1), jnp.float32),
            pltpu.SemaphoreType.DMA,
            pltpu.SemaphoreType.DMA,
            pltpu.SemaphoreType.DMA,
            pltpu.SemaphoreType.DMA,
        ],
    )
    def k(idxt_hbm, table_hbm, out_hbm, idx_v, gbuf, wbuf, gs0, gs1, ws0, ws1):
        gsems = (gs0, gs1)
        wsems = (ws0, ws1)
        wid = lax.axis_index("s") * _NC + lax.axis_index("c")

        pltpu.sync_copy(idxt_hbm.at[:, pl.ds(wid * BLK, BLK)], idx_v)

        def fire_gather(t, b):
            pltpu.async_copy(table_hbm.at[idx_v.at[t]], gbuf.at[b], gsems[b])

        def fire_writes(t, b):
            for db in range(_DB):
                pltpu.async_copy(
                    wbuf.at[b, pl.ds(db * _SUB, _SUB), pl.ds(0, BLK)],
                    out_hbm.at[(t * _DB + db) * _NW + wid],
                    wsems[b],
                )

        def wait_writes(b):
            for _ in range(_DB):
                pltpu.make_async_copy(
                    wbuf.at[0, pl.ds(0, _SUB), pl.ds(0, BLK)],
                    out_hbm.at[0],
                    wsems[b],
                ).wait()

        for b in range(NBUF):
            fire_gather(b, b)

        lanes = lax.iota(jnp.int32, 16)

        def body(t0, carry):
            for b in range(NBUF):
                t = t0 + b
                @pl.when(t0 >= NBUF)
                def _():
                    wait_writes(b)

                pltpu.make_async_copy(
                    table_hbm.at[idx_v.at[0]], gbuf.at[b], gsems[b]
                ).wait()

                @plsc.parallel_loop(0, BLK, unroll=8)
                def _(tok):
                    svec = jnp.full((16,), tok, jnp.int32)
                    for c in (0, 16):
                        vals = gbuf[b, tok, pl.ds(c, 16)] * SCALE
                        plsc.store_scatter(
                            wbuf.at[b], [lanes + c, svec], vals
                        )

                fire_writes(t, b)

                @pl.when(t0 + NBUF < seq_len)
                def _():
                    fire_gather(t + NBUF, b)
            return carry

        lax.fori_loop(0, seq_len // NBUF,
                      lambda i, cr: body(i * NBUF, cr), 0, unroll=False)

        for b in range(NBUF):
            wait_writes(b)

    return k


def kernel(sequences, table):
    n_seq, seq_len = sequences.shape
    idxt = sequences.T
    out_tiles = _make_sc_kernel(n_seq, seq_len)(idxt, table)
    out = out_tiles.reshape(seq_len, _DB, _NW, _SUB, BLK)
    out = out.transpose(2, 4, 0, 1, 3)
    return out.reshape(n_seq, seq_len, D_MODEL)


# speedup vs baseline: 4.0930x; 2.0026x over previous
import functools

import jax
import jax.numpy as jnp
import numpy as np
from jax import lax
from jax.experimental import pallas as pl
from jax.experimental.pallas import tpu as pltpu
from jax.experimental.pallas import tpu_sc as plsc

D_MODEL = 32
BLK = 128
NBUF = 2
SCALE = np.float32(np.sqrt(np.float32(D_MODEL)))

_NC = 2
_NS = 16
_NW = _NC * _NS
_SUB = 8
_DB = D_MODEL // _SUB


VOCAB_TILE = 128


def _make_detile_kernel(v_size: int):
    n_full = v_size // VOCAB_TILE
    tail = v_size - n_full * VOCAB_TILE
    per_w = (n_full + _NW - 1) // _NW
    elems = VOCAB_TILE * D_MODEL
    PITCH = VOCAB_TILE + 9

    mesh = plsc.VectorSubcoreMesh(core_axis_name="c", subcore_axis_name="s")

    @functools.partial(
        pl.kernel,
        mesh=mesh,
        out_type=jax.ShapeDtypeStruct((v_size * D_MODEL,), jnp.float32),
        compiler_params=pltpu.CompilerParams(
            use_tc_tiling_on_sc=True, needs_layout_passes=False
        ),
        scratch_types=[
            pltpu.VMEM((D_MODEL, VOCAB_TILE), jnp.float32),
            pltpu.VMEM((D_MODEL, VOCAB_TILE), jnp.float32),
            pltpu.VMEM((D_MODEL, tail), jnp.float32),
            pltpu.VMEM((D_MODEL * PITCH,), jnp.float32),
            pltpu.VMEM((D_MODEL * PITCH,), jnp.float32),
            pltpu.VMEM((elems,), jnp.float32),
            pltpu.VMEM((elems,), jnp.float32),
            pltpu.SemaphoreType.DMA,
            pltpu.SemaphoreType.DMA,
            pltpu.SemaphoreType.DMA,
            pltpu.SemaphoreType.DMA,
        ],
    )
    def k(tabt_hbm, tail_hbm, out_hbm, tbuf0, tbuf1, tbuf2, pbuf0, pbuf1,
          obuf0, obuf1, rs0, rs1, ws0, ws1):
        tbufs = (tbuf0, tbuf1)
        pbufs = (pbuf0, pbuf1)
        obufs = (obuf0, obuf1)
        rsems = (rs0, rs1)
        wsems = (ws0, ws1)
        wid = lax.axis_index("s") * _NC + lax.axis_index("c")
        c_lo = wid * per_w
        c_hi = jnp.minimum(c_lo + per_w, n_full)

        def fire_read(c, b):
            pltpu.async_copy(
                tabt_hbm.at[:, pl.ds(c * VOCAB_TILE, VOCAB_TILE)],
                tbufs[b], rsems[b],
            )

        def wait_read(b):
            pltpu.make_async_copy(
                tabt_hbm.at[:, pl.ds(0, VOCAB_TILE)], tbufs[b], rsems[b]
            ).wait()

        def wait_write(b):
            pltpu.make_async_copy(
                obufs[b], out_hbm.at[pl.ds(0, elems)], wsems[b]
            ).wait()

        lanes = lax.iota(jnp.int32, 16)
        gbase = lanes * PITCH

        def transpose(src, b, n_tok):
            @plsc.parallel_loop(0, D_MODEL, unroll=4)
            def _(d):
                for j in range(0, n_tok, 16):
                    vals = src[d, pl.ds(j, 16)]
                    plsc.store_scatter(
                        pbufs[b], [lanes + (d * PITCH + j)], vals
                    )

            @plsc.parallel_loop(0, n_tok, unroll=8)
            def _(v):
                for c16 in (0, 16):
                    vals = plsc.load_gather(
                        pbufs[b], [gbase + (c16 * PITCH + v)]
                    )
                    obufs[b][pl.ds(v * D_MODEL + c16, 16)] = vals

        for b in range(NBUF):
            @pl.when(c_lo + b < c_hi)
            def _():
                fire_read(c_lo + b, b)

        def body(i, carry):
            for b in range(NBUF):
                c = c_lo + i + b

                @pl.when(c < c_hi)
                def _():
                    wait_read(b)

                    @pl.when(i > 0)
                    def _():
                        wait_write(b)

                    transpose(tbufs[b], b, VOCAB_TILE)
                    pltpu.async_copy(
                        obufs[b],
                        out_hbm.at[pl.ds(c * elems, elems)],
                        wsems[b],
                    )

                    @pl.when(c + NBUF < c_hi)
                    def _():
                        fire_read(c + NBUF, b)
            return carry

        lax.fori_loop(0, (per_w + NBUF - 1) // NBUF,
                      lambda i, cr: body(i * NBUF, cr), 0, unroll=False)

        for b in range(NBUF):
            @pl.when(c_lo + b < c_hi)
            def _():
                wait_write(b)

        if tail:
            @pl.when(wid == _NW - 1)
            def _():
                pltpu.sync_copy(tail_hbm, tbuf2)
                transpose(tbuf2, 0, tail)
                pltpu.sync_copy(
                    obuf0.at[pl.ds(0, tail * D_MODEL)],
                    out_hbm.at[pl.ds(n_full * elems, tail * D_MODEL)],
                )

    return k


def _make_sc_kernel(n_seq: int, seq_len: int):
    assert n_seq == _NW * BLK
    n_tiles = seq_len * _DB * (n_seq // BLK)

    mesh = plsc.VectorSubcoreMesh(core_axis_name="c", subcore_axis_name="s")

    @functools.partial(
        pl.kernel,
        mesh=mesh,
        out_type=jax.ShapeDtypeStruct((n_tiles, _SUB, BLK), jnp.float32),
        compiler_params=pltpu.CompilerParams(
            use_tc_tiling_on_sc=False, needs_layout_passes=False
        ),
        scratch_types=[
            pltpu.VMEM((seq_len, BLK), jnp.int32),
            pltpu.VMEM((NBUF, BLK, D_MODEL), jnp.float32),
            pltpu.VMEM((NBUF, D_MODEL, BLK + 1), jnp.float32),
            pltpu.SemaphoreType.DMA,
            pltpu.SemaphoreType.DMA,
            pltpu.SemaphoreType.DMA,
            pltpu.SemaphoreType.DMA,
        ],
    )
    def k(idxt_hbm, table_hbm, out_hbm, idx_v, gbuf, wbuf, gs0, gs1, ws0, ws1):
        gsems = (gs0, gs1)
        wsems = (ws0, ws1)
        wid = lax.axis_index("s") * _NC + lax.axis_index("c")

        pltpu.sync_copy(idxt_hbm.at[:, pl.ds(wid * BLK, BLK)], idx_v)

        def fire_gather(t, b):
            pltpu.async_copy(table_hbm.at[idx_v.at[t]], gbuf.at[b], gsems[b])

        def fire_writes(t, b):
            for db in range(_DB):
                pltpu.async_copy(
                    wbuf.at[b, pl.ds(db * _SUB, _SUB), pl.ds(0, BLK)],
                    out_hbm.at[(t * _DB + db) * _NW + wid],
                    wsems[b],
                )

        def wait_writes(b):
            for _ in range(_DB):
                pltpu.make_async_copy(
                    wbuf.at[0, pl.ds(0, _SUB), pl.ds(0, BLK)],
                    out_hbm.at[0],
                    wsems[b],
                ).wait()

        for b in range(NBUF):
            fire_gather(b, b)

        lanes = lax.iota(jnp.int32, 16)

        def body(t0, carry):
            for b in range(NBUF):
                t = t0 + b
                @pl.when(t0 >= NBUF)
                def _():
                    wait_writes(b)

                pltpu.make_async_copy(
                    table_hbm.at[idx_v.at[0]], gbuf.at[b], gsems[b]
                ).wait()

                @plsc.parallel_loop(0, BLK, unroll=8)
                def _(tok):
                    svec = jnp.full((16,), tok, jnp.int32)
                    for c in (0, 16):
                        vals = gbuf[b, tok, pl.ds(c, 16)] * SCALE
                        plsc.store_scatter(
                            wbuf.at[b], [lanes + c, svec], vals
                        )

                fire_writes(t, b)

                @pl.when(t0 + NBUF < seq_len)
                def _():
                    fire_gather(t + NBUF, b)
            return carry

        lax.fori_loop(0, seq_len // NBUF,
                      lambda i, cr: body(i * NBUF, cr), 0, unroll=False)

        for b in range(NBUF):
            wait_writes(b)

    return k


def kernel(sequences, table):
    n_seq, seq_len = sequences.shape
    v_size = table.shape[0]
    idxt = sequences.T
    tail = v_size % VOCAB_TILE
    tail_t = table[v_size - tail:].T
    table_lin = _make_detile_kernel(v_size)(table.T, tail_t)
    table_rm = table_lin.reshape(v_size, D_MODEL)
    out_tiles = _make_sc_kernel(n_seq, seq_len)(idxt, table_rm)
    out = out_tiles.reshape(seq_len, _DB, _NW, _SUB, BLK)
    out = out.transpose(2, 4, 0, 1, 3)
    return out.reshape(n_seq, seq_len, D_MODEL)


# speedup vs baseline: 5.1207x; 1.2511x over previous
import functools

import jax
import jax.numpy as jnp
import numpy as np
from jax import lax
from jax.experimental import pallas as pl
from jax.experimental.pallas import tpu as pltpu
from jax.experimental.pallas import tpu_sc as plsc

D_MODEL = 32
BLK = 128
NBUF = 2
SCALE = np.float32(np.sqrt(np.float32(D_MODEL)))

_NC = 2
_NS = 16
_NW = _NC * _NS
_SUB = 8
_DB = D_MODEL // _SUB


VOCAB_TILE = 512


def _make_detile_kernel(v_size: int):
    n_full = v_size // VOCAB_TILE
    tail = v_size - n_full * VOCAB_TILE
    per_w = (n_full + _NW - 1) // _NW
    elems = VOCAB_TILE * D_MODEL
    PITCH = VOCAB_TILE + 9

    mesh = plsc.VectorSubcoreMesh(core_axis_name="c", subcore_axis_name="s")

    @functools.partial(
        pl.kernel,
        mesh=mesh,
        out_type=jax.ShapeDtypeStruct((v_size * D_MODEL,), jnp.float32),
        compiler_params=pltpu.CompilerParams(
            use_tc_tiling_on_sc=True, needs_layout_passes=False
        ),
        scratch_types=[
            pltpu.VMEM((D_MODEL, VOCAB_TILE), jnp.float32),
            pltpu.VMEM((D_MODEL, VOCAB_TILE), jnp.float32),
            pltpu.VMEM((D_MODEL, tail), jnp.float32),
            pltpu.VMEM((D_MODEL * PITCH,), jnp.float32),
            pltpu.VMEM((D_MODEL * PITCH,), jnp.float32),
            pltpu.VMEM((elems,), jnp.float32),
            pltpu.VMEM((elems,), jnp.float32),
            pltpu.SemaphoreType.DMA,
            pltpu.SemaphoreType.DMA,
            pltpu.SemaphoreType.DMA,
            pltpu.SemaphoreType.DMA,
        ],
    )
    def k(tabt_hbm, tail_hbm, out_hbm, tbuf0, tbuf1, tbuf2, pbuf0, pbuf1,
          obuf0, obuf1, rs0, rs1, ws0, ws1):
        tbufs = (tbuf0, tbuf1)
        pbufs = (pbuf0, pbuf1)
        obufs = (obuf0, obuf1)
        rsems = (rs0, rs1)
        wsems = (ws0, ws1)
        wid = lax.axis_index("s") * _NC + lax.axis_index("c")
        c_lo = wid * per_w
        c_hi = jnp.minimum(c_lo + per_w, n_full)

        def fire_read(c, b):
            pltpu.async_copy(
                tabt_hbm.at[:, pl.ds(c * VOCAB_TILE, VOCAB_TILE)],
                tbufs[b], rsems[b],
            )

        def wait_read(b):
            pltpu.make_async_copy(
                tabt_hbm.at[:, pl.ds(0, VOCAB_TILE)], tbufs[b], rsems[b]
            ).wait()

        def wait_write(b):
            pltpu.make_async_copy(
                obufs[b], out_hbm.at[pl.ds(0, elems)], wsems[b]
            ).wait()

        lanes = lax.iota(jnp.int32, 16)
        gbase = lanes * PITCH

        def transpose(src, b, n_tok):
            @plsc.parallel_loop(0, D_MODEL, unroll=4)
            def _(d):
                for j in range(0, n_tok, 16):
                    vals = src[d, pl.ds(j, 16)]
                    plsc.store_scatter(
                        pbufs[b], [lanes + (d * PITCH + j)], vals
                    )

            @plsc.parallel_loop(0, n_tok, unroll=8)
            def _(v):
                for c16 in (0, 16):
                    vals = plsc.load_gather(
                        pbufs[b], [gbase + (c16 * PITCH + v)]
                    )
                    obufs[b][pl.ds(v * D_MODEL + c16, 16)] = vals

        for b in range(NBUF):
            @pl.when(c_lo + b < c_hi)
            def _():
                fire_read(c_lo + b, b)

        def body(i, carry):
            for b in range(NBUF):
                c = c_lo + i + b

                @pl.when(c < c_hi)
                def _():
                    wait_read(b)

                    @pl.when(i > 0)
                    def _():
                        wait_write(b)

                    transpose(tbufs[b], b, VOCAB_TILE)
                    pltpu.async_copy(
                        obufs[b],
                        out_hbm.at[pl.ds(c * elems, elems)],
                        wsems[b],
                    )

                    @pl.when(c + NBUF < c_hi)
                    def _():
                        fire_read(c + NBUF, b)
            return carry

        lax.fori_loop(0, (per_w + NBUF - 1) // NBUF,
                      lambda i, cr: body(i * NBUF, cr), 0, unroll=False)

        for b in range(NBUF):
            @pl.when(c_lo + b < c_hi)
            def _():
                wait_write(b)

        if tail:
            @pl.when(wid == _NW - 1)
            def _():
                pltpu.sync_copy(tail_hbm, tbuf2)
                transpose(tbuf2, 0, tail)
                pltpu.sync_copy(
                    obuf0.at[pl.ds(0, tail * D_MODEL)],
                    out_hbm.at[pl.ds(n_full * elems, tail * D_MODEL)],
                )

    return k


def _make_sc_kernel(n_seq: int, seq_len: int):
    assert n_seq == _NW * BLK
    n_tiles = seq_len * _DB * (n_seq // BLK)

    mesh = plsc.VectorSubcoreMesh(core_axis_name="c", subcore_axis_name="s")

    @functools.partial(
        pl.kernel,
        mesh=mesh,
        out_type=jax.ShapeDtypeStruct((n_tiles, _SUB, BLK), jnp.float32),
        compiler_params=pltpu.CompilerParams(
            use_tc_tiling_on_sc=False, needs_layout_passes=False
        ),
        scratch_types=[
            pltpu.VMEM((seq_len, BLK), jnp.int32),
            pltpu.VMEM((NBUF, BLK, D_MODEL), jnp.float32),
            pltpu.VMEM((NBUF, D_MODEL, BLK + 1), jnp.float32),
            pltpu.SemaphoreType.DMA,
            pltpu.SemaphoreType.DMA,
            pltpu.SemaphoreType.DMA,
            pltpu.SemaphoreType.DMA,
        ],
    )
    def k(idxt_hbm, table_hbm, out_hbm, idx_v, gbuf, wbuf, gs0, gs1, ws0, ws1):
        gsems = (gs0, gs1)
        wsems = (ws0, ws1)
        wid = lax.axis_index("s") * _NC + lax.axis_index("c")

        pltpu.sync_copy(idxt_hbm.at[:, pl.ds(wid * BLK, BLK)], idx_v)

        def fire_gather(t, b):
            pltpu.async_copy(table_hbm.at[idx_v.at[t]], gbuf.at[b], gsems[b])

        def fire_writes(t, b):
            for db in range(_DB):
                pltpu.async_copy(
                    wbuf.at[b, pl.ds(db * _SUB, _SUB), pl.ds(0, BLK)],
                    out_hbm.at[(t * _DB + db) * _NW + wid],
                    wsems[b],
                )

        def wait_writes(b):
            for _ in range(_DB):
                pltpu.make_async_copy(
                    wbuf.at[0, pl.ds(0, _SUB), pl.ds(0, BLK)],
                    out_hbm.at[0],
                    wsems[b],
                ).wait()

        for b in range(NBUF):
            fire_gather(b, b)

        lanes = lax.iota(jnp.int32, 16)

        def body(t0, carry):
            for b in range(NBUF):
                t = t0 + b
                @pl.when(t0 >= NBUF)
                def _():
                    wait_writes(b)

                pltpu.make_async_copy(
                    table_hbm.at[idx_v.at[0]], gbuf.at[b], gsems[b]
                ).wait()

                @plsc.parallel_loop(0, BLK, unroll=8)
                def _(tok):
                    svec = jnp.full((16,), tok, jnp.int32)
                    for c in (0, 16):
                        vals = gbuf[b, tok, pl.ds(c, 16)] * SCALE
                        plsc.store_scatter(
                            wbuf.at[b], [lanes + c, svec], vals
                        )

                fire_writes(t, b)

                @pl.when(t0 + NBUF < seq_len)
                def _():
                    fire_gather(t + NBUF, b)
            return carry

        lax.fori_loop(0, seq_len // NBUF,
                      lambda i, cr: body(i * NBUF, cr), 0, unroll=False)

        for b in range(NBUF):
            wait_writes(b)

    return k


def kernel(sequences, table):
    n_seq, seq_len = sequences.shape
    v_size = table.shape[0]
    idxt = sequences.T
    tail = v_size % VOCAB_TILE
    tail_t = table[v_size - tail:].T
    table_lin = _make_detile_kernel(v_size)(table.T, tail_t)
    table_rm = table_lin.reshape(v_size, D_MODEL)
    out_tiles = _make_sc_kernel(n_seq, seq_len)(idxt, table_rm)
    out = out_tiles.reshape(seq_len, _DB, _NW, _SUB, BLK)
    out = out.transpose(2, 4, 0, 1, 3)
    return out.reshape(n_seq, seq_len, D_MODEL)


# speedup vs baseline: 5.1979x; 1.0151x over previous
import functools

import jax
import jax.numpy as jnp
import numpy as np
from jax import lax
from jax.experimental import pallas as pl
from jax.experimental.pallas import tpu as pltpu
from jax.experimental.pallas import tpu_sc as plsc

D_MODEL = 32
BLK = 128
NBUF = 2
SCALE = np.float32(np.sqrt(np.float32(D_MODEL)))

_NC = 2
_NS = 16
_NW = _NC * _NS
_SUB = 8
_DB = D_MODEL // _SUB


VOCAB_TILE = 512


def _make_detile_kernel(v_size: int):
    n_full = v_size // VOCAB_TILE
    tail = v_size - n_full * VOCAB_TILE
    per_w = (n_full + _NW - 1) // _NW
    elems = VOCAB_TILE * D_MODEL
    PITCH = VOCAB_TILE + 9

    mesh = plsc.VectorSubcoreMesh(core_axis_name="c", subcore_axis_name="s")

    @functools.partial(
        pl.kernel,
        mesh=mesh,
        out_type=jax.ShapeDtypeStruct((v_size * D_MODEL,), jnp.float32),
        compiler_params=pltpu.CompilerParams(
            use_tc_tiling_on_sc=True, needs_layout_passes=False
        ),
        scratch_types=[
            pltpu.VMEM((D_MODEL, VOCAB_TILE), jnp.float32),
            pltpu.VMEM((D_MODEL, VOCAB_TILE), jnp.float32),
            pltpu.VMEM((D_MODEL, tail), jnp.float32),
            pltpu.VMEM((D_MODEL * PITCH,), jnp.float32),
            pltpu.VMEM((D_MODEL * PITCH,), jnp.float32),
            pltpu.VMEM((elems,), jnp.float32),
            pltpu.VMEM((elems,), jnp.float32),
            pltpu.SemaphoreType.DMA,
            pltpu.SemaphoreType.DMA,
            pltpu.SemaphoreType.DMA,
            pltpu.SemaphoreType.DMA,
        ],
    )
    def k(tabt_hbm, tail_hbm, out_hbm, tbuf0, tbuf1, tbuf2, pbuf0, pbuf1,
          obuf0, obuf1, rs0, rs1, ws0, ws1):
        tbufs = (tbuf0, tbuf1)
        pbufs = (pbuf0, pbuf1)
        obufs = (obuf0, obuf1)
        rsems = (rs0, rs1)
        wsems = (ws0, ws1)
        wid = lax.axis_index("s") * _NC + lax.axis_index("c")
        c_lo = wid * per_w
        c_hi = jnp.minimum(c_lo + per_w, n_full)

        def fire_read(c, b):
            pltpu.async_copy(
                tabt_hbm.at[:, pl.ds(c * VOCAB_TILE, VOCAB_TILE)],
                tbufs[b], rsems[b],
            )

        def wait_read(b):
            pltpu.make_async_copy(
                tabt_hbm.at[:, pl.ds(0, VOCAB_TILE)], tbufs[b], rsems[b]
            ).wait()

        def wait_write(b):
            pltpu.make_async_copy(
                obufs[b], out_hbm.at[pl.ds(0, elems)], wsems[b]
            ).wait()

        lanes = lax.iota(jnp.int32, 16)
        gbase = lanes * PITCH

        def transpose(src, b, n_tok):
            @plsc.parallel_loop(0, D_MODEL, unroll=4)
            def _(d):
                for j in range(0, n_tok, 16):
                    vals = src[d, pl.ds(j, 16)]
                    plsc.store_scatter(
                        pbufs[b], [lanes + (d * PITCH + j)], vals
                    )

            @plsc.parallel_loop(0, n_tok, unroll=8)
            def _(v):
                for c16 in (0, 16):
                    vals = plsc.load_gather(
                        pbufs[b], [gbase + (c16 * PITCH + v)]
                    )
                    obufs[b][pl.ds(v * D_MODEL + c16, 16)] = vals * SCALE

        for b in range(NBUF):
            @pl.when(c_lo + b < c_hi)
            def _():
                fire_read(c_lo + b, b)

        def body(i, carry):
            for b in range(NBUF):
                c = c_lo + i + b

                @pl.when(c < c_hi)
                def _():
                    wait_read(b)

                    @pl.when(i > 0)
                    def _():
                        wait_write(b)

                    transpose(tbufs[b], b, VOCAB_TILE)
                    pltpu.async_copy(
                        obufs[b],
                        out_hbm.at[pl.ds(c * elems, elems)],
                        wsems[b],
                    )

                    @pl.when(c + NBUF < c_hi)
                    def _():
                        fire_read(c + NBUF, b)
            return carry

        lax.fori_loop(0, (per_w + NBUF - 1) // NBUF,
                      lambda i, cr: body(i * NBUF, cr), 0, unroll=False)

        for b in range(NBUF):
            @pl.when(c_lo + b < c_hi)
            def _():
                wait_write(b)

        if tail:
            @pl.when(wid == _NW - 1)
            def _():
                pltpu.sync_copy(tail_hbm, tbuf2)
                transpose(tbuf2, 0, tail)
                pltpu.sync_copy(
                    obuf0.at[pl.ds(0, tail * D_MODEL)],
                    out_hbm.at[pl.ds(n_full * elems, tail * D_MODEL)],
                )

    return k


def _make_sc_kernel(n_seq: int, seq_len: int):
    assert n_seq == _NW * BLK
    n_tiles = seq_len * _DB * (n_seq // BLK)
    mesh = plsc.VectorSubcoreMesh(core_axis_name="c", subcore_axis_name="s")

    @functools.partial(
        pl.kernel,
        mesh=mesh,
        out_type=jax.ShapeDtypeStruct((n_tiles, _SUB, BLK), jnp.float32),
        compiler_params=pltpu.CompilerParams(
            use_tc_tiling_on_sc=False, needs_layout_passes=False
        ),
        scratch_types=[
            pltpu.VMEM((seq_len, BLK), jnp.int32),
            pltpu.VMEM((NBUF, BLK, D_MODEL), jnp.float32),
            pltpu.VMEM((NBUF, D_MODEL, BLK + 1), jnp.float32),
            pltpu.SemaphoreType.DMA,
            pltpu.SemaphoreType.DMA,
            pltpu.SemaphoreType.DMA,
            pltpu.SemaphoreType.DMA,
        ],
    )
    def k(idxt_hbm, table_hbm, out_hbm, idx_v, gbuf, wbuf, gs0, gs1, ws0, ws1):
        gsems = (gs0, gs1)
        wsems = (ws0, ws1)
        wid = lax.axis_index("s") * _NC + lax.axis_index("c")

        pltpu.sync_copy(idxt_hbm.at[:, pl.ds(wid * BLK, BLK)], idx_v)

        def fire_gather(t, b):
            pltpu.async_copy(table_hbm.at[idx_v.at[t]], gbuf.at[b], gsems[b])

        def fire_writes(t, b):
            for db in range(_DB):
                pltpu.async_copy(
                    wbuf.at[b, pl.ds(db * _SUB, _SUB), pl.ds(0, BLK)],
                    out_hbm.at[(t * _DB + db) * _NW + wid],
                    wsems[b],
                )

        def wait_writes(b):
            for _ in range(_DB):
                pltpu.make_async_copy(
                    wbuf.at[0, pl.ds(0, _SUB), pl.ds(0, BLK)],
                    out_hbm.at[0],
                    wsems[b],
                ).wait()

        for b in range(NBUF):
            fire_gather(b, b)

        lanes = lax.iota(jnp.int32, 16)

        def body(t0, carry):
            for b in range(NBUF):
                t = t0 + b
                @pl.when(t0 >= NBUF)
                def _():
                    wait_writes(b)

                pltpu.make_async_copy(
                    table_hbm.at[idx_v.at[0]], gbuf.at[b], gsems[b]
                ).wait()

                @plsc.parallel_loop(0, BLK, unroll=16)
                def _(tok):
                    svec = jnp.full((16,), tok, jnp.int32)
                    for c in (0, 16):
                        vals = gbuf[b, tok, pl.ds(c, 16)]
                        plsc.store_scatter(
                            wbuf.at[b], [lanes + c, svec], vals
                        )

                fire_writes(t, b)

                @pl.when(t0 + NBUF < seq_len)
                def _():
                    fire_gather(t + NBUF, b)
            return carry

        lax.fori_loop(0, seq_len // NBUF,
                      lambda i, cr: body(i * NBUF, cr), 0, unroll=False)

        for b in range(NBUF):
            wait_writes(b)

    return k


def kernel(sequences, table):
    n_seq, seq_len = sequences.shape
    v_size = table.shape[0]
    idxt = sequences.T
    tail = v_size % VOCAB_TILE
    tail_t = table[v_size - tail:].T
    table_lin = _make_detile_kernel(v_size)(table.T, tail_t)
    table_rm = table_lin.reshape(v_size, D_MODEL)
    out_tiles = _make_sc_kernel(n_seq, seq_len)(idxt, table_rm)
    out = out_tiles.reshape(seq_len, _DB, _NW, _SUB, BLK)
    out = out.transpose(2, 4, 0, 1, 3)
    return out.reshape(n_seq, seq_len, D_MODEL)


# speedup vs baseline: 6.3203x; 1.2159x over previous
import functools

import jax
import jax.numpy as jnp
import numpy as np
from jax import lax
from jax.experimental import pallas as pl
from jax.experimental.pallas import tpu as pltpu
from jax.experimental.pallas import tpu_sc as plsc

D_MODEL = 32
BLK = 128
NBUF = 2
SCALE = np.float32(np.sqrt(np.float32(D_MODEL)))

_NC = 2
_NS = 16
_NW = _NC * _NS
_SUB = 8
_DB = D_MODEL // _SUB


VOCAB_TILE = 512


def _make_detile_kernel(v_size: int):
    n_full = v_size // VOCAB_TILE
    tail = v_size - n_full * VOCAB_TILE
    per_w = (n_full + _NW - 1) // _NW
    elems = VOCAB_TILE * D_MODEL
    PITCH = VOCAB_TILE + 9

    mesh = plsc.VectorSubcoreMesh(core_axis_name="c", subcore_axis_name="s")

    @functools.partial(
        pl.kernel,
        mesh=mesh,
        out_type=jax.ShapeDtypeStruct((v_size * D_MODEL,), jnp.float32),
        compiler_params=pltpu.CompilerParams(
            use_tc_tiling_on_sc=True, needs_layout_passes=False
        ),
        scratch_types=[
            pltpu.VMEM((D_MODEL, VOCAB_TILE), jnp.float32),
            pltpu.VMEM((D_MODEL, VOCAB_TILE), jnp.float32),
            pltpu.VMEM((D_MODEL, tail), jnp.float32),
            pltpu.VMEM((D_MODEL * PITCH,), jnp.float32),
            pltpu.VMEM((D_MODEL * PITCH,), jnp.float32),
            pltpu.VMEM((elems,), jnp.float32),
            pltpu.VMEM((elems,), jnp.float32),
            pltpu.SemaphoreType.DMA,
            pltpu.SemaphoreType.DMA,
            pltpu.SemaphoreType.DMA,
            pltpu.SemaphoreType.DMA,
        ],
    )
    def k(tabt_hbm, tail_hbm, out_hbm, tbuf0, tbuf1, tbuf2, pbuf0, pbuf1,
          obuf0, obuf1, rs0, rs1, ws0, ws1):
        tbufs = (tbuf0, tbuf1)
        pbufs = (pbuf0, pbuf1)
        obufs = (obuf0, obuf1)
        rsems = (rs0, rs1)
        wsems = (ws0, ws1)
        wid = lax.axis_index("s") * _NC + lax.axis_index("c")
        c_lo = wid * per_w
        c_hi = jnp.minimum(c_lo + per_w, n_full)

        def fire_read(c, b):
            pltpu.async_copy(
                tabt_hbm.at[:, pl.ds(c * VOCAB_TILE, VOCAB_TILE)],
                tbufs[b], rsems[b],
            )

        def wait_read(b):
            pltpu.make_async_copy(
                tabt_hbm.at[:, pl.ds(0, VOCAB_TILE)], tbufs[b], rsems[b]
            ).wait()

        def wait_write(b):
            pltpu.make_async_copy(
                obufs[b], out_hbm.at[pl.ds(0, elems)], wsems[b]
            ).wait()

        lanes = lax.iota(jnp.int32, 16)
        gbase = lanes * PITCH

        def transpose(src, b, n_tok):
            @plsc.parallel_loop(0, D_MODEL, unroll=4)
            def _(d):
                for j in range(0, n_tok, 16):
                    vals = src[d, pl.ds(j, 16)]
                    plsc.store_scatter(
                        pbufs[b], [lanes + (d * PITCH + j)], vals
                    )

            @plsc.parallel_loop(0, n_tok, unroll=8)
            def _(v):
                for c16 in (0, 16):
                    vals = plsc.load_gather(
                        pbufs[b], [gbase + (c16 * PITCH + v)]
                    )
                    obufs[b][pl.ds(v * D_MODEL + c16, 16)] = vals * SCALE

        for b in range(NBUF):
            @pl.when(c_lo + b < c_hi)
            def _():
                fire_read(c_lo + b, b)

        def body(i, carry):
            for b in range(NBUF):
                c = c_lo + i + b

                @pl.when(c < c_hi)
                def _():
                    wait_read(b)

                    @pl.when(i > 0)
                    def _():
                        wait_write(b)

                    transpose(tbufs[b], b, VOCAB_TILE)
                    pltpu.async_copy(
                        obufs[b],
                        out_hbm.at[pl.ds(c * elems, elems)],
                        wsems[b],
                    )

                    @pl.when(c + NBUF < c_hi)
                    def _():
                        fire_read(c + NBUF, b)
            return carry

        lax.fori_loop(0, (per_w + NBUF - 1) // NBUF,
                      lambda i, cr: body(i * NBUF, cr), 0, unroll=False)

        for b in range(NBUF):
            @pl.when(c_lo + b < c_hi)
            def _():
                wait_write(b)

        if tail:
            @pl.when(wid == _NW - 1)
            def _():
                pltpu.sync_copy(tail_hbm, tbuf2)
                transpose(tbuf2, 0, tail)
                pltpu.sync_copy(
                    obuf0.at[pl.ds(0, tail * D_MODEL)],
                    out_hbm.at[pl.ds(n_full * elems, tail * D_MODEL)],
                )

    return k


GB_N = 4


def _make_sc_kernel(n_seq: int, seq_len: int):
    assert n_seq == _NW * BLK
    assert seq_len % GB_N == 0
    n_tiles = seq_len * _DB * (n_seq // BLK)
    mesh = plsc.VectorSubcoreMesh(core_axis_name="c", subcore_axis_name="s")

    @functools.partial(
        pl.kernel,
        mesh=mesh,
        out_type=jax.ShapeDtypeStruct((n_tiles, _SUB, BLK), jnp.float32),
        compiler_params=pltpu.CompilerParams(
            use_tc_tiling_on_sc=False, needs_layout_passes=False
        ),
        scratch_types=[
            pltpu.VMEM((seq_len, BLK), jnp.int32),
            pltpu.VMEM((GB_N, BLK, D_MODEL), jnp.float32),
            pltpu.VMEM((GB_N, D_MODEL, BLK + 1), jnp.float32),
        ] + [pltpu.SemaphoreType.DMA] * (2 * GB_N),
    )
    def k(idxt_hbm, table_hbm, out_hbm, idx_v, gbuf, wbuf, *sems):
        gsems = sems[:GB_N]
        wsems = sems[GB_N:]
        wid = lax.axis_index("s") * _NC + lax.axis_index("c")

        pltpu.sync_copy(idxt_hbm.at[:, pl.ds(wid * BLK, BLK)], idx_v)

        def fire_gather(t, b):
            pltpu.async_copy(table_hbm.at[idx_v.at[t]], gbuf.at[b], gsems[b])

        def fire_writes(t, b):
            for db in range(_DB):
                pltpu.async_copy(
                    wbuf.at[b, pl.ds(db * _SUB, _SUB), pl.ds(0, BLK)],
                    out_hbm.at[(t * _DB + db) * _NW + wid],
                    wsems[b],
                )

        def wait_writes(b):
            for _ in range(_DB):
                pltpu.make_async_copy(
                    wbuf.at[0, pl.ds(0, _SUB), pl.ds(0, BLK)],
                    out_hbm.at[0],
                    wsems[b],
                ).wait()

        for b in range(GB_N):
            fire_gather(b, b)

        lanes = lax.iota(jnp.int32, 16)

        def body(t0, carry):
            for b in range(GB_N):
                t = t0 + b
                @pl.when(t0 >= GB_N)
                def _():
                    wait_writes(b)

                pltpu.make_async_copy(
                    table_hbm.at[idx_v.at[0]], gbuf.at[b], gsems[b]
                ).wait()

                @plsc.parallel_loop(0, BLK, unroll=16)
                def _(tok):
                    svec = jnp.full((16,), tok, jnp.int32)
                    for c in (0, 16):
                        vals = gbuf[b, tok, pl.ds(c, 16)]
                        plsc.store_scatter(
                            wbuf.at[b], [lanes + c, svec], vals
                        )

                fire_writes(t, b)

                @pl.when(t0 + GB_N < seq_len)
                def _():
                    fire_gather(t + GB_N, b)
            return carry

        lax.fori_loop(0, seq_len // GB_N,
                      lambda i, cr: body(i * GB_N, cr), 0, unroll=False)

        for b in range(GB_N):
            wait_writes(b)

    return k


def kernel(sequences, table):
    n_seq, seq_len = sequences.shape
    v_size = table.shape[0]
    idxt = sequences.T
    tail = v_size % VOCAB_TILE
    tail_t = table[v_size - tail:].T
    table_lin = _make_detile_kernel(v_size)(table.T, tail_t)
    table_rm = table_lin.reshape(v_size, D_MODEL)
    out_tiles = _make_sc_kernel(n_seq, seq_len)(idxt, table_rm)
    out = out_tiles.reshape(seq_len, _DB, _NW, _SUB, BLK)
    out = out.transpose(2, 4, 0, 1, 3)
    return out.reshape(n_seq, seq_len, D_MODEL)


# speedup vs baseline: 6.3511x; 1.0049x over previous
import functools

import jax
import jax.numpy as jnp
import numpy as np
from jax import lax
from jax.experimental import pallas as pl
from jax.experimental.pallas import tpu as pltpu
from jax.experimental.pallas import tpu_sc as plsc

D_MODEL = 32
BLK = 128
NBUF = 2
SCALE = np.float32(np.sqrt(np.float32(D_MODEL)))

_NC = 2
_NS = 16
_NW = _NC * _NS
_SUB = 8
_DB = D_MODEL // _SUB


VOCAB_TILE = 512


def _make_detile_kernel(v_size: int):
    n_full = v_size // VOCAB_TILE
    tail = v_size - n_full * VOCAB_TILE
    per_w = (n_full + _NW - 1) // _NW
    elems = VOCAB_TILE * D_MODEL
    PITCH = VOCAB_TILE + 9

    mesh = plsc.VectorSubcoreMesh(core_axis_name="c", subcore_axis_name="s")

    @functools.partial(
        pl.kernel,
        mesh=mesh,
        out_type=jax.ShapeDtypeStruct((v_size * D_MODEL,), jnp.float32),
        compiler_params=pltpu.CompilerParams(
            use_tc_tiling_on_sc=True, needs_layout_passes=False
        ),
        scratch_types=[
            pltpu.VMEM((D_MODEL, VOCAB_TILE), jnp.float32),
            pltpu.VMEM((D_MODEL, VOCAB_TILE), jnp.float32),
            pltpu.VMEM((D_MODEL, tail), jnp.float32),
            pltpu.VMEM((D_MODEL * PITCH,), jnp.float32),
            pltpu.VMEM((D_MODEL * PITCH,), jnp.float32),
            pltpu.VMEM((elems,), jnp.float32),
            pltpu.VMEM((elems,), jnp.float32),
            pltpu.SemaphoreType.DMA,
            pltpu.SemaphoreType.DMA,
            pltpu.SemaphoreType.DMA,
            pltpu.SemaphoreType.DMA,
        ],
    )
    def k(tabt_hbm, tail_hbm, out_hbm, tbuf0, tbuf1, tbuf2, pbuf0, pbuf1,
          obuf0, obuf1, rs0, rs1, ws0, ws1):
        tbufs = (tbuf0, tbuf1)
        pbufs = (pbuf0, pbuf1)
        obufs = (obuf0, obuf1)
        rsems = (rs0, rs1)
        wsems = (ws0, ws1)
        wid = lax.axis_index("s") * _NC + lax.axis_index("c")
        c_lo = wid * per_w
        c_hi = jnp.minimum(c_lo + per_w, n_full)

        def fire_read(c, b):
            pltpu.async_copy(
                tabt_hbm.at[:, pl.ds(c * VOCAB_TILE, VOCAB_TILE)],
                tbufs[b], rsems[b],
            )

        def wait_read(b):
            pltpu.make_async_copy(
                tabt_hbm.at[:, pl.ds(0, VOCAB_TILE)], tbufs[b], rsems[b]
            ).wait()

        def wait_write(b):
            pltpu.make_async_copy(
                obufs[b], out_hbm.at[pl.ds(0, elems)], wsems[b]
            ).wait()

        lanes = lax.iota(jnp.int32, 16)
        gbase = lanes * PITCH

        def transpose(src, b, n_tok):
            @plsc.parallel_loop(0, D_MODEL, unroll=4)
            def _(d):
                for j in range(0, n_tok, 16):
                    vals = src[d, pl.ds(j, 16)]
                    plsc.store_scatter(
                        pbufs[b], [lanes + (d * PITCH + j)], vals
                    )

            @plsc.parallel_loop(0, n_tok, unroll=8)
            def _(v):
                for c16 in (0, 16):
                    vals = plsc.load_gather(
                        pbufs[b], [gbase + (c16 * PITCH + v)]
                    )
                    obufs[b][pl.ds(v * D_MODEL + c16, 16)] = vals * SCALE

        for b in range(NBUF):
            @pl.when(c_lo + b < c_hi)
            def _():
                fire_read(c_lo + b, b)

        def body(i, carry):
            for b in range(NBUF):
                c = c_lo + i + b

                @pl.when(c < c_hi)
                def _():
                    wait_read(b)

                    @pl.when(i > 0)
                    def _():
                        wait_write(b)

                    transpose(tbufs[b], b, VOCAB_TILE)
                    pltpu.async_copy(
                        obufs[b],
                        out_hbm.at[pl.ds(c * elems, elems)],
                        wsems[b],
                    )

                    @pl.when(c + NBUF < c_hi)
                    def _():
                        fire_read(c + NBUF, b)
            return carry

        lax.fori_loop(0, (per_w + NBUF - 1) // NBUF,
                      lambda i, cr: body(i * NBUF, cr), 0, unroll=False)

        for b in range(NBUF):
            @pl.when(c_lo + b < c_hi)
            def _():
                wait_write(b)

        if tail:
            @pl.when(wid == _NW - 1)
            def _():
                pltpu.sync_copy(tail_hbm, tbuf2)
                transpose(tbuf2, 0, tail)
                pltpu.sync_copy(
                    obuf0.at[pl.ds(0, tail * D_MODEL)],
                    out_hbm.at[pl.ds(n_full * elems, tail * D_MODEL)],
                )

    return k


GB_N = 8


def _make_sc_kernel(n_seq: int, seq_len: int):
    assert n_seq == _NW * BLK
    assert seq_len % GB_N == 0
    n_tiles = seq_len * _DB * (n_seq // BLK)
    mesh = plsc.VectorSubcoreMesh(core_axis_name="c", subcore_axis_name="s")

    @functools.partial(
        pl.kernel,
        mesh=mesh,
        out_type=jax.ShapeDtypeStruct((n_tiles, _SUB, BLK), jnp.float32),
        compiler_params=pltpu.CompilerParams(
            use_tc_tiling_on_sc=False, needs_layout_passes=False
        ),
        scratch_types=[
            pltpu.VMEM((seq_len, BLK), jnp.int32),
            pltpu.VMEM((GB_N, BLK, D_MODEL), jnp.float32),
            pltpu.VMEM((GB_N, D_MODEL, BLK + 1), jnp.float32),
        ] + [pltpu.SemaphoreType.DMA] * (2 * GB_N),
    )
    def k(idxt_hbm, table_hbm, out_hbm, idx_v, gbuf, wbuf, *sems):
        gsems = sems[:GB_N]
        wsems = sems[GB_N:]
        wid = lax.axis_index("s") * _NC + lax.axis_index("c")

        pltpu.sync_copy(idxt_hbm.at[:, pl.ds(wid * BLK, BLK)], idx_v)

        def fire_gather(t, b):
            pltpu.async_copy(table_hbm.at[idx_v.at[t]], gbuf.at[b], gsems[b])

        def fire_writes(t, b):
            for db in range(_DB):
                pltpu.async_copy(
                    wbuf.at[b, pl.ds(db * _SUB, _SUB), pl.ds(0, BLK)],
                    out_hbm.at[(t * _DB + db) * _NW + wid],
                    wsems[b],
                )

        def wait_writes(b):
            for _ in range(_DB):
                pltpu.make_async_copy(
                    wbuf.at[0, pl.ds(0, _SUB), pl.ds(0, BLK)],
                    out_hbm.at[0],
                    wsems[b],
                ).wait()

        for b in range(GB_N):
            fire_gather(b, b)

        lanes = lax.iota(jnp.int32, 16)

        def body(t0, carry):
            for b in range(GB_N):
                t = t0 + b
                @pl.when(t0 >= GB_N)
                def _():
                    wait_writes(b)

                pltpu.make_async_copy(
                    table_hbm.at[idx_v.at[0]], gbuf.at[b], gsems[b]
                ).wait()

                @plsc.parallel_loop(0, BLK, unroll=16)
                def _(tok):
                    svec = jnp.full((16,), tok, jnp.int32)
                    for c in (0, 16):
                        vals = gbuf[b, tok, pl.ds(c, 16)]
                        plsc.store_scatter(
                            wbuf.at[b], [lanes + c, svec], vals
                        )

                fire_writes(t, b)

                @pl.when(t0 + GB_N < seq_len)
                def _():
                    fire_gather(t + GB_N, b)
            return carry

        lax.fori_loop(0, seq_len // GB_N,
                      lambda i, cr: body(i * GB_N, cr), 0, unroll=False)

        for b in range(GB_N):
            wait_writes(b)

    return k


def kernel(sequences, table):
    n_seq, seq_len = sequences.shape
    v_size = table.shape[0]
    idxt = sequences.T
    tail = v_size % VOCAB_TILE
    tail_t = table[v_size - tail:].T
    table_lin = _make_detile_kernel(v_size)(table.T, tail_t)
    table_rm = table_lin.reshape(v_size, D_MODEL)
    out_tiles = _make_sc_kernel(n_seq, seq_len)(idxt, table_rm)
    out = out_tiles.reshape(seq_len, _DB, _NW, _SUB, BLK)
    out = out.transpose(2, 4, 0, 1, 3)
    return out.reshape(n_seq, seq_len, D_MODEL)
